# Initial kernel scaffold; baseline (speedup 1.0000x reference)
import jax
import jax.numpy as jnp
from jax.experimental import pallas as pl


def kernel(inputs, W_s, b_s, W_flr, b_flr, W_out, b_out):
    raise NotImplementedError("write your pallas kernel here")


# speedup vs baseline: 10.5606x; 10.5606x over previous
import jax
import jax.numpy as jnp
from jax import lax
from jax.experimental import pallas as pl

B, V, F_IN = 2, 4096, 64
K, S_DIM, F_LR, F_OUT = 16, 4, 12, 18
TI = 512

_HI = lax.Precision.HIGHEST


def _knn_kernel(xt_ref, xT_ref, ws_ref, bs_ref, wsT_ref, bsc_ref,
                wfT_ref, bfc_ref, wo1_ref, wo2_ref, wo3_ref, bo_ref,
                out_ref):
    i = pl.program_id(1)
    XT = xT_ref[0]
    ST = jnp.dot(wsT_ref[...], XT, precision=_HI) + bsc_ref[...]
    FT = jnp.dot(wfT_ref[...], XT, precision=_HI) + bfc_ref[...]

    Xt = xt_ref[0]
    St = jnp.dot(Xt, ws_ref[...], precision=_HI) + bs_ref[...]

    d2 = jnp.zeros((TI, V), jnp.float32)
    for s in range(S_DIM):
        diff = St[:, s:s + 1] - ST[s:s + 1, :]
        d2 = d2 + diff * diff

    col = lax.broadcasted_iota(jnp.int32, (TI, V), 1)
    row = i * TI + lax.broadcasted_iota(jnp.int32, (TI, V), 0)
    d2 = jnp.where(col == row, jnp.inf, d2)

    def body(_, work):
        m = jnp.min(work, axis=1, keepdims=True)
        return jnp.where(work == m, jnp.inf, work)
    work = lax.fori_loop(0, K - 1, body, d2)
    t = jnp.min(work, axis=1, keepdims=True)

    mask = d2 <= t
    w = jnp.where(mask, jnp.exp(-10.0 * d2), 0.0)

    agg_mean = lax.dot_general(w, FT, (((1,), (1,)), ((), ())),
                               preferred_element_type=jnp.float32,
                               precision=_HI) / K

    neg_inf = jnp.float32(-jnp.inf)
    cols = []
    for f in range(F_LR):
        vals = jnp.where(mask, w * FT[f:f + 1, :], neg_inf)
        cols.append(jnp.max(vals, axis=1, keepdims=True))
    agg_max = jnp.concatenate(cols, axis=1)

    acc = (jnp.dot(Xt, wo1_ref[...], precision=_HI)
           + jnp.dot(agg_mean, wo2_ref[...], precision=_HI)
           + jnp.dot(agg_max, wo3_ref[...], precision=_HI))
    out_ref[0] = jnp.tanh(acc + bo_ref[...])


def kernel(inputs, W_s, b_s, W_flr, b_flr, W_out, b_out):
    Wo1 = W_out[:F_IN]
    Wo2 = W_out[F_IN:F_IN + F_LR]
    Wo3 = W_out[F_IN + F_LR:]
    inputs_T = jnp.swapaxes(inputs, 1, 2)
    grid = (B, V // TI)
    full = lambda b, i: (b, 0, 0)
    return pl.pallas_call(
        _knn_kernel,
        grid=grid,
        in_specs=[
            pl.BlockSpec((1, TI, F_IN), lambda b, i: (b, i, 0)),
            pl.BlockSpec((1, F_IN, V), full),
            pl.BlockSpec((F_IN, S_DIM), lambda b, i: (0, 0)),
            pl.BlockSpec((1, S_DIM), lambda b, i: (0, 0)),
            pl.BlockSpec((S_DIM, F_IN), lambda b, i: (0, 0)),
            pl.BlockSpec((S_DIM, 1), lambda b, i: (0, 0)),
            pl.BlockSpec((F_LR, F_IN), lambda b, i: (0, 0)),
            pl.BlockSpec((F_LR, 1), lambda b, i: (0, 0)),
            pl.BlockSpec((F_IN, F_OUT), lambda b, i: (0, 0)),
            pl.BlockSpec((F_LR, F_OUT), lambda b, i: (0, 0)),
            pl.BlockSpec((F_LR, F_OUT), lambda b, i: (0, 0)),
            pl.BlockSpec((1, F_OUT), lambda b, i: (0, 0)),
        ],
        out_specs=pl.BlockSpec((1, TI, F_OUT), lambda b, i: (b, i, 0)),
        out_shape=jax.ShapeDtypeStruct((B, V, F_OUT), jnp.float32),
    )(inputs, inputs_T, W_s, b_s[None, :], W_s.T, b_s[:, None],
      W_flr.T, b_flr[:, None], Wo1, Wo2, Wo3, b_out[None, :])
